# Initial kernel scaffold; baseline (speedup 1.0000x reference)
import jax
import jax.numpy as jnp
from jax.experimental import pallas as pl


def kernel(x, W):
    raise NotImplementedError("write your pallas kernel here")


# speedup vs baseline: 1.8454x; 1.8454x over previous
import functools

import jax
import jax.numpy as jnp
from jax import lax
from jax.experimental import pallas as pl
from jax.experimental.pallas import tpu as pltpu
from jax.experimental.pallas import tpu_sc as plsc

_INFO = plsc.get_sparse_core_info()
_NC = _INFO.num_cores
_NS = _INFO.num_subcores
_NW = _NC * _NS

_B = 16384 * 50
_D = 64
_IW = 128
_NROWS = _B // _IW
_ROWS_PER_W = _NROWS // _NW
_CHUNK_ROWS = 8
_STEPS = _ROWS_PER_W // _CHUNK_ROWS
_CHUNK = _CHUNK_ROWS * _IW


@functools.partial(
    pl.kernel,
    mesh=plsc.VectorSubcoreMesh(core_axis_name="c", subcore_axis_name="s"),
    out_type=jax.ShapeDtypeStruct((_B, _D), jnp.float32),
    scratch_types=[
        pltpu.VMEM((_CHUNK_ROWS, _IW), jnp.int32),
        pltpu.VMEM((_CHUNK, _D), jnp.float32),
        pltpu.SemaphoreType.DMA,
    ],
    compiler_params=pltpu.CompilerParams(use_tc_tiling_on_sc=False),
)
def _gather_kernel(idx_hbm, table_hbm, out_hbm, idx_v, rows_v, sem):
    wid = lax.axis_index("s") * _NC + lax.axis_index("c")
    base_row = wid * _ROWS_PER_W

    def step(i, carry):
        row0 = base_row + i * _CHUNK_ROWS
        pltpu.sync_copy(idx_hbm.at[pl.ds(row0, _CHUNK_ROWS)], idx_v)
        copies = [
            pltpu.async_copy(
                table_hbm.at[idx_v.at[j]],
                rows_v.at[pl.ds(j * _IW, _IW)],
                sem,
            )
            for j in range(_CHUNK_ROWS)
        ]
        for c in copies:
            c.wait()
        pltpu.sync_copy(rows_v, out_hbm.at[pl.ds(row0 * _IW, _CHUNK)])
        return carry

    lax.fori_loop(0, _STEPS, step, 0)


def kernel(x, W):
    idx = x.reshape(_NROWS, _IW).astype(jnp.int32)
    out = _gather_kernel(idx, W)
    return out.reshape(x.shape + (W.shape[1],))
